# Initial kernel scaffold; baseline (speedup 1.0000x reference)
import jax
import jax.numpy as jnp
from jax.experimental import pallas as pl


def kernel(depth, feat, ranks_depth, ranks_feat, maxn):
    raise NotImplementedError("write your pallas kernel here")


# speedup vs baseline: 11.3102x; 11.3102x over previous
import functools

import jax
import jax.numpy as jnp
from jax import lax
from jax.experimental import pallas as pl
from jax.experimental.pallas import tpu as pltpu
from jax.experimental.pallas import tpu_sc as plsc

_BEV_FEAT_SHAPE = (1, 1, 200, 200, 64)
_NC, _NS, _LANES = 2, 16, 16
_NW = _NC * _NS
_IC = 128
_CC = 64


@functools.lru_cache(maxsize=None)
def _make_bevpool(n_cells, mx, C, p1, f1):
    del p1, f1
    kc = C // _LANES
    G = _CC * mx
    n_idx_rows = G // _IC
    total_chunks = n_cells // _CC

    mesh = plsc.VectorSubcoreMesh(
        core_axis_name="c", subcore_axis_name="s",
        num_cores=_NC, num_subcores=_NS)

    @functools.partial(
        pl.kernel,
        out_type=jax.ShapeDtypeStruct((n_cells, C), jnp.float32),
        mesh=mesh,
        compiler_params=pltpu.CompilerParams(use_tc_tiling_on_sc=False),
        scratch_types=[
            pltpu.VMEM((n_idx_rows, _IC), jnp.int32),
            pltpu.VMEM((n_idx_rows, _IC), jnp.int32),
            pltpu.VMEM((n_idx_rows, _IC), jnp.float32),
            pltpu.VMEM((G, C), jnp.float32),
            pltpu.VMEM((_CC, C), jnp.float32),
            pltpu.SemaphoreType.DMA,
            pltpu.SemaphoreType.DMA,
        ],
    )
    def bev_kernel(depth_hbm, feat_hbm, rd_hbm, rf_hbm, out_hbm,
                   idxd_v, idxf_v, dvals_v, frows_v, obuf_v, sem_d, sem_f):
        wid = lax.axis_index("s") * _NC + lax.axis_index("c")
        my_chunks = (total_chunks - wid + _NW - 1) // _NW

        def chunk_body(t, _):
            c = wid + t * _NW
            cell0 = c * _CC
            pltpu.sync_copy(rd_hbm.at[pl.ds(c * n_idx_rows, n_idx_rows)], idxd_v)
            pltpu.sync_copy(rf_hbm.at[pl.ds(c * n_idx_rows, n_idx_rows)], idxf_v)
            copies = []
            for i in range(n_idx_rows):
                copies.append(pltpu.async_copy(
                    depth_hbm.at[idxd_v.at[i]], dvals_v.at[i], sem_d))
                copies.append(pltpu.async_copy(
                    feat_hbm.at[idxf_v.at[i]],
                    frows_v.at[pl.ds(i * _IC, _IC)], sem_f))
            for cp in copies:
                cp.wait()

            def cell_body(j, _):
                pb = j * mx
                row = j // (_IC // mx)
                col0 = (j % (_IC // mx)) * mx
                acc = [jnp.zeros((_LANES,), jnp.float32) for _ in range(kc)]
                dvec = dvals_v[row, pl.ds(col0, mx)]
                for p in range(mx):
                    d = dvec[p]
                    for k in range(kc):
                        acc[k] = acc[k] + d * frows_v[pb + p,
                                                      pl.ds(k * _LANES, _LANES)]
                for k in range(kc):
                    obuf_v[j, pl.ds(k * _LANES, _LANES)] = acc[k]
                return 0

            lax.fori_loop(0, _CC, cell_body, 0)
            pltpu.sync_copy(obuf_v, out_hbm.at[pl.ds(cell0, _CC)])
            return 0

        lax.fori_loop(0, my_chunks, chunk_body, 0)

    return bev_kernel


def kernel(depth, feat, ranks_depth, ranks_feat, maxn):
    del maxn
    C = feat.shape[-1]
    _, oD, oW, oH, _ = _BEV_FEAT_SHAPE
    n_cells = oD * oW * oH
    L = ranks_depth.shape[0]
    mx = L // n_cells
    depth_flat = jnp.concatenate(
        [depth.reshape(-1), jnp.zeros((1,), jnp.float32)])
    feat_2d = jnp.concatenate(
        [feat.reshape(-1, C), jnp.zeros((1, C), jnp.float32)], axis=0)
    fn = _make_bevpool(n_cells, mx, C, depth_flat.shape[0], feat_2d.shape[0])
    out = fn(depth_flat, feat_2d,
             ranks_depth.reshape(-1, _IC), ranks_feat.reshape(-1, _IC))
    return out.reshape(1, oD, oW, oH, C)


# speedup vs baseline: 13.8224x; 1.2221x over previous
import functools

import jax
import jax.numpy as jnp
from jax import lax
from jax.experimental import pallas as pl
from jax.experimental.pallas import tpu as pltpu
from jax.experimental.pallas import tpu_sc as plsc

_BEV_FEAT_SHAPE = (1, 1, 200, 200, 64)
_NC, _NS, _LANES = 2, 16, 16
_NW = _NC * _NS
_IC = 80
_CC = 25


@functools.lru_cache(maxsize=None)
def _make_bevpool(n_cells, mx, C, p1, f1):
    del p1, f1
    kc = C // _LANES
    G = _CC * mx
    n_str = G // _IC
    cells_per_w = n_cells // _NW
    n_chunks = cells_per_w // _CC
    idx_rows = cells_per_w * mx // _IC
    n_pairs = n_chunks // 2

    mesh = plsc.VectorSubcoreMesh(
        core_axis_name="c", subcore_axis_name="s",
        num_cores=_NC, num_subcores=_NS)

    @functools.partial(
        pl.kernel,
        out_type=jax.ShapeDtypeStruct((n_cells, C), jnp.float32),
        mesh=mesh,
        compiler_params=pltpu.CompilerParams(use_tc_tiling_on_sc=False),
        scratch_types=[
            pltpu.VMEM((idx_rows, _IC), jnp.int32),
            pltpu.VMEM((idx_rows, _IC), jnp.int32),
            pltpu.VMEM((G,), jnp.float32),
            pltpu.VMEM((G,), jnp.float32),
            pltpu.VMEM((G, C), jnp.float32),
            pltpu.VMEM((G, C), jnp.float32),
            pltpu.VMEM((_CC, C), jnp.float32),
            pltpu.VMEM((_CC, C), jnp.float32),
            pltpu.SemaphoreType.DMA,
            pltpu.SemaphoreType.DMA,
        ],
    )
    def bev_kernel(depth_hbm, feat_hbm, rd_hbm, rf_hbm, out_hbm,
                   rdx_v, rfx_v, dvA, dvB, frA, frB, obA, obB,
                   sem_g, sem_o):
        wid = lax.axis_index("s") * _NC + lax.axis_index("c")
        cell_base = wid * cells_per_w
        row_base = wid * idx_rows

        pltpu.sync_copy(rd_hbm.at[pl.ds(row_base, idx_rows)], rdx_v)
        pltpu.sync_copy(rf_hbm.at[pl.ds(row_base, idx_rows)], rfx_v)

        def fire_gathers(t, dv, fr):
            for i in range(n_str):
                r = t * n_str + i
                pltpu.async_copy(
                    depth_hbm.at[rdx_v.at[r]],
                    dv.at[pl.ds(i * _IC, _IC)], sem_g)
                pltpu.async_copy(
                    feat_hbm.at[rfx_v.at[r]],
                    fr.at[pl.ds(i * _IC, _IC)], sem_g)

        def drain_gathers(dv, fr):
            for i in range(n_str):
                pltpu.make_async_copy(
                    depth_hbm.at[rdx_v.at[i]],
                    dv.at[pl.ds(i * _IC, _IC)], sem_g).wait()
                pltpu.make_async_copy(
                    feat_hbm.at[rfx_v.at[i]],
                    fr.at[pl.ds(i * _IC, _IC)], sem_g).wait()

        def compute_chunk(t, dv, fr, ob):
            def cell_body(j, _):
                pb = j * mx
                dvec = dv[pl.ds(pb, mx)]
                acc = [jnp.zeros((_LANES,), jnp.float32) for _ in range(kc)]
                for p in range(mx):
                    d = dvec[p]
                    for k in range(kc):
                        acc[k] = acc[k] + d * fr[pb + p,
                                                 pl.ds(k * _LANES, _LANES)]
                for k in range(kc):
                    ob[j, pl.ds(k * _LANES, _LANES)] = acc[k]
                return 0

            lax.fori_loop(0, _CC, cell_body, 0)
            pltpu.async_copy(
                ob, out_hbm.at[pl.ds(cell_base + t * _CC, _CC)], sem_o)

        def drain_out(ob):
            pltpu.make_async_copy(
                ob, out_hbm.at[pl.ds(cell_base, _CC)], sem_o).wait()

        def step(t, dv, fr, ob, dv_n, fr_n, has_next, may_drain_out):
            drain_gathers(dv, fr)

            if has_next is None:
                fire_gathers(t + 1, dv_n, fr_n)
            else:
                @pl.when(has_next)
                def _():
                    fire_gathers(t + 1, dv_n, fr_n)

            @pl.when(may_drain_out)
            def _():
                drain_out(ob)

            compute_chunk(t, dv, fr, ob)

        fire_gathers(0, dvA, frA)

        def pair_body(q, _):
            t0 = 2 * q
            step(t0, dvA, frA, obA, dvB, frB,
                 has_next=None, may_drain_out=q >= 1)
            step(t0 + 1, dvB, frB, obB, dvA, frA,
                 has_next=q < n_pairs - 1, may_drain_out=q >= 1)
            return 0

        lax.fori_loop(0, n_pairs, pair_body, 0)
        drain_out(obA)
        drain_out(obB)

    return bev_kernel


def kernel(depth, feat, ranks_depth, ranks_feat, maxn):
    del maxn
    C = feat.shape[-1]
    _, oD, oW, oH, _ = _BEV_FEAT_SHAPE
    n_cells = oD * oW * oH
    L = ranks_depth.shape[0]
    mx = L // n_cells
    depth_flat = jnp.concatenate(
        [depth.reshape(-1), jnp.zeros((1,), jnp.float32)])
    feat_2d = jnp.concatenate(
        [feat.reshape(-1, C), jnp.zeros((1, C), jnp.float32)], axis=0)
    fn = _make_bevpool(n_cells, mx, C, depth_flat.shape[0], feat_2d.shape[0])
    out = fn(depth_flat, feat_2d,
             ranks_depth.reshape(-1, _IC), ranks_feat.reshape(-1, _IC))
    return out.reshape(1, oD, oW, oH, C)


# speedup vs baseline: 15.9454x; 1.1536x over previous
import functools

import jax
import jax.numpy as jnp
from jax import lax
from jax.experimental import pallas as pl
from jax.experimental.pallas import tpu as pltpu
from jax.experimental.pallas import tpu_sc as plsc

_BEV_FEAT_SHAPE = (1, 1, 200, 200, 64)
_NC, _NS, _LANES = 2, 16, 16
_NW = _NC * _NS
_IC = 80
_CC = 25


@functools.lru_cache(maxsize=None)
def _make_bevpool(n_cells, mx, C, p1, f1):
    del p1, f1
    kc = C // _LANES
    G = _CC * mx
    n_str = G // _IC
    cells_per_w = n_cells // _NW
    n_chunks = cells_per_w // _CC
    idx_rows = cells_per_w * mx // _IC
    n_pairs = n_chunks // 2

    mesh = plsc.VectorSubcoreMesh(
        core_axis_name="c", subcore_axis_name="s",
        num_cores=_NC, num_subcores=_NS)

    @functools.partial(
        pl.kernel,
        out_type=jax.ShapeDtypeStruct((n_cells, C), jnp.float32),
        mesh=mesh,
        compiler_params=pltpu.CompilerParams(
            use_tc_tiling_on_sc=False, needs_layout_passes=False),
        scratch_types=[
            pltpu.VMEM((idx_rows, _IC), jnp.int32),
            pltpu.VMEM((idx_rows, _IC), jnp.int32),
            pltpu.VMEM((G,), jnp.float32),
            pltpu.VMEM((G,), jnp.float32),
            pltpu.VMEM((G, C), jnp.bfloat16),
            pltpu.VMEM((G, C), jnp.bfloat16),
            pltpu.VMEM((_CC, C), jnp.float32),
            pltpu.VMEM((_CC, C), jnp.float32),
            pltpu.SemaphoreType.DMA,
            pltpu.SemaphoreType.DMA,
        ],
    )
    def bev_kernel(depth_hbm, feat_hbm, rd_hbm, rf_hbm, out_hbm,
                   rdx_v, rfx_v, dvA, dvB, frA, frB, obA, obB,
                   sem_g, sem_o):
        wid = lax.axis_index("s") * _NC + lax.axis_index("c")
        cell_base = wid * cells_per_w
        row_base = wid * idx_rows

        pltpu.sync_copy(rd_hbm.at[pl.ds(row_base, idx_rows)], rdx_v)
        pltpu.sync_copy(rf_hbm.at[pl.ds(row_base, idx_rows)], rfx_v)

        def fire_gathers(t, dv, fr):
            for i in range(n_str):
                r = t * n_str + i
                pltpu.async_copy(
                    depth_hbm.at[rdx_v.at[r]],
                    dv.at[pl.ds(i * _IC, _IC)], sem_g)
                pltpu.async_copy(
                    feat_hbm.at[rfx_v.at[r]],
                    fr.at[pl.ds(i * _IC, _IC)], sem_g)

        def drain_gathers(dv, fr):
            for i in range(n_str):
                pltpu.make_async_copy(
                    depth_hbm.at[rdx_v.at[i]],
                    dv.at[pl.ds(i * _IC, _IC)], sem_g).wait()
                pltpu.make_async_copy(
                    feat_hbm.at[rfx_v.at[i]],
                    fr.at[pl.ds(i * _IC, _IC)], sem_g).wait()

        def compute_chunk(t, dv, fr, ob):
            def cell_body(j, _):
                pb = j * mx
                dvec = dv[pl.ds(pb, mx)]
                acc = [jnp.zeros((_LANES,), jnp.float32) for _ in range(kc)]
                for p in range(mx):
                    d = dvec[p]
                    for k2 in range(kc // 2):
                        packed = fr[pb + p, pl.ds(k2 * 2 * _LANES, 2 * _LANES)]
                        lo, hi = plsc.unpack(
                            packed, format=plsc.PackFormat.INTERLEAVED)
                        acc[2 * k2] = acc[2 * k2] + d * lo
                        acc[2 * k2 + 1] = acc[2 * k2 + 1] + d * hi
                for k in range(kc):
                    ob[j, pl.ds(k * _LANES, _LANES)] = acc[k]
                return 0

            lax.fori_loop(0, _CC, cell_body, 0)
            pltpu.async_copy(
                ob, out_hbm.at[pl.ds(cell_base + t * _CC, _CC)], sem_o)

        def drain_out(ob):
            pltpu.make_async_copy(
                ob, out_hbm.at[pl.ds(cell_base, _CC)], sem_o).wait()

        def step(t, dv, fr, ob, dv_n, fr_n, has_next, may_drain_out):
            drain_gathers(dv, fr)

            if has_next is None:
                fire_gathers(t + 1, dv_n, fr_n)
            else:
                @pl.when(has_next)
                def _():
                    fire_gathers(t + 1, dv_n, fr_n)

            @pl.when(may_drain_out)
            def _():
                drain_out(ob)

            compute_chunk(t, dv, fr, ob)

        fire_gathers(0, dvA, frA)

        def pair_body(q, _):
            t0 = 2 * q
            step(t0, dvA, frA, obA, dvB, frB,
                 has_next=None, may_drain_out=q >= 1)
            step(t0 + 1, dvB, frB, obB, dvA, frA,
                 has_next=q < n_pairs - 1, may_drain_out=q >= 1)
            return 0

        lax.fori_loop(0, n_pairs, pair_body, 0)
        drain_out(obA)
        drain_out(obB)

    return bev_kernel


def kernel(depth, feat, ranks_depth, ranks_feat, maxn):
    del maxn
    C = feat.shape[-1]
    _, oD, oW, oH, _ = _BEV_FEAT_SHAPE
    n_cells = oD * oW * oH
    L = ranks_depth.shape[0]
    mx = L // n_cells
    depth_flat = jnp.concatenate(
        [depth.reshape(-1), jnp.zeros((1,), jnp.float32)])
    feat_2d = jnp.concatenate(
        [feat.reshape(-1, C), jnp.zeros((1, C), jnp.float32)], axis=0)
    f1 = feat_2d.shape[0]
    feat_bf = (feat_2d.reshape(f1, C // 32, 2, 16)
               .swapaxes(2, 3).reshape(f1, C).astype(jnp.bfloat16))
    fn = _make_bevpool(n_cells, mx, C, depth_flat.shape[0], f1)
    out = fn(depth_flat, feat_bf,
             ranks_depth.reshape(-1, _IC), ranks_feat.reshape(-1, _IC))
    return out.reshape(1, oD, oW, oH, C)


# speedup vs baseline: 17.9814x; 1.1277x over previous
import functools

import jax
import jax.numpy as jnp
from jax import lax
from jax.experimental import pallas as pl
from jax.experimental.pallas import tpu as pltpu
from jax.experimental.pallas import tpu_sc as plsc

_BEV_FEAT_SHAPE = (1, 1, 200, 200, 64)
_NC, _NS, _LANES = 2, 16, 16
_NW = _NC * _NS
_IC = 80
_CC = 25
_NBUF = 4


@functools.lru_cache(maxsize=None)
def _make_bevpool(n_cells, mx, C, p1, f1):
    del p1, f1
    kc = C // _LANES
    G = _CC * mx
    n_str = G // _IC
    cells_per_w = n_cells // _NW
    n_chunks = cells_per_w // _CC
    idx_rows = cells_per_w * mx // _IC

    mesh = plsc.VectorSubcoreMesh(
        core_axis_name="c", subcore_axis_name="s",
        num_cores=_NC, num_subcores=_NS)

    scratch = ([pltpu.VMEM((idx_rows, _IC), jnp.int32)] * 2
               + [pltpu.VMEM((G,), jnp.float32)] * _NBUF
               + [pltpu.VMEM((G, C), jnp.bfloat16)] * _NBUF
               + [pltpu.VMEM((_CC, C), jnp.float32)] * 2
               + [pltpu.SemaphoreType.DMA] * 2)

    @functools.partial(
        pl.kernel,
        out_type=jax.ShapeDtypeStruct((n_cells, C), jnp.float32),
        mesh=mesh,
        compiler_params=pltpu.CompilerParams(
            use_tc_tiling_on_sc=False, needs_layout_passes=False),
        scratch_types=scratch,
    )
    def bev_kernel(depth_hbm, feat_hbm, rd_hbm, rf_hbm, out_hbm,
                   rdx_v, rfx_v, dv0, dv1, dv2, dv3, fr0, fr1, fr2, fr3,
                   ob0, ob1, sem_g, sem_o):
        dvs = (dv0, dv1, dv2, dv3)
        frs = (fr0, fr1, fr2, fr3)
        obs = (ob0, ob1)
        wid = lax.axis_index("s") * _NC + lax.axis_index("c")
        cell_base = wid * cells_per_w
        row_base = wid * idx_rows

        pltpu.sync_copy(rd_hbm.at[pl.ds(row_base, idx_rows)], rdx_v)
        pltpu.sync_copy(rf_hbm.at[pl.ds(row_base, idx_rows)], rfx_v)

        def fire_gathers(t, dv, fr):
            for i in range(n_str):
                r = t * n_str + i
                pltpu.async_copy(
                    depth_hbm.at[rdx_v.at[r]],
                    dv.at[pl.ds(i * _IC, _IC)], sem_g)
                pltpu.async_copy(
                    feat_hbm.at[rfx_v.at[r]],
                    fr.at[pl.ds(i * _IC, _IC)], sem_g)

        def drain_gathers(dv, fr):
            for i in range(n_str):
                pltpu.make_async_copy(
                    depth_hbm.at[rdx_v.at[i]],
                    dv.at[pl.ds(i * _IC, _IC)], sem_g).wait()
                pltpu.make_async_copy(
                    feat_hbm.at[rfx_v.at[i]],
                    fr.at[pl.ds(i * _IC, _IC)], sem_g).wait()

        def compute_chunk(t, dv, fr, ob):
            def cell_body(j, _):
                pb = j * mx
                dvec = dv[pl.ds(pb, mx)]
                dd = plsc.bitcast(
                    plsc.pack(dvec, dvec, format=plsc.PackFormat.INTERLEAVED),
                    jnp.int32)
                acc = [jnp.zeros((_LANES,), jnp.float32) for _ in range(kc)]
                for p in range(mx):
                    dsplat = plsc.bitcast(
                        jnp.broadcast_to(dd[p], (_LANES,)), jnp.bfloat16)
                    for k2 in range(kc // 2):
                        packed = fr[pb + p, pl.ds(k2 * 2 * _LANES, 2 * _LANES)]
                        lo, hi = plsc.unpack(
                            dsplat * packed, format=plsc.PackFormat.INTERLEAVED)
                        acc[2 * k2] = acc[2 * k2] + lo
                        acc[2 * k2 + 1] = acc[2 * k2 + 1] + hi
                for k in range(kc):
                    ob[j, pl.ds(k * _LANES, _LANES)] = acc[k]
                return 0

            lax.fori_loop(0, _CC, cell_body, 0)
            pltpu.async_copy(
                ob, out_hbm.at[pl.ds(cell_base + t * _CC, _CC)], sem_o)

        def drain_out(ob):
            pltpu.make_async_copy(
                ob, out_hbm.at[pl.ds(cell_base, _CC)], sem_o).wait()

        fire_gathers(0, dvs[0], frs[0])
        fire_gathers(1, dvs[1], frs[1])

        def quad_body(q, _):
            for u in range(_NBUF):
                t = _NBUF * q + u
                fire_gathers(t + 2, dvs[(u + 2) % _NBUF], frs[(u + 2) % _NBUF])
                drain_gathers(dvs[u], frs[u])
                if u >= 2:
                    drain_out(obs[u % 2])
                else:
                    @pl.when(t >= 2)
                    def _():
                        drain_out(obs[u % 2])
                compute_chunk(t, dvs[u], frs[u], obs[u % 2])
            return 0

        lax.fori_loop(0, (n_chunks - 2) // _NBUF, quad_body, 0)
        for tt in (n_chunks - 2, n_chunks - 1):
            u = tt % _NBUF
            drain_gathers(dvs[u], frs[u])
            drain_out(obs[tt % 2])
            compute_chunk(tt, dvs[u], frs[u], obs[tt % 2])
        drain_out(obs[0])
        drain_out(obs[1])

    return bev_kernel


def kernel(depth, feat, ranks_depth, ranks_feat, maxn):
    del maxn
    C = feat.shape[-1]
    _, oD, oW, oH, _ = _BEV_FEAT_SHAPE
    n_cells = oD * oW * oH
    L = ranks_depth.shape[0]
    mx = L // n_cells
    depth_flat = jnp.concatenate(
        [depth.reshape(-1), jnp.zeros((1,), jnp.float32)])
    feat_2d = jnp.concatenate(
        [feat.reshape(-1, C), jnp.zeros((1, C), jnp.float32)], axis=0)
    f1 = feat_2d.shape[0]
    feat_bf = (feat_2d.reshape(f1, C // 32, 2, 16)
               .swapaxes(2, 3).reshape(f1, C).astype(jnp.bfloat16))
    fn = _make_bevpool(n_cells, mx, C, depth_flat.shape[0], f1)
    out = fn(depth_flat, feat_bf,
             ranks_depth.reshape(-1, _IC), ranks_feat.reshape(-1, _IC))
    return out.reshape(1, oD, oW, oH, C)


# speedup vs baseline: 17.9822x; 1.0000x over previous
import functools

import jax
import jax.numpy as jnp
from jax import lax
from jax.experimental import pallas as pl
from jax.experimental.pallas import tpu as pltpu
from jax.experimental.pallas import tpu_sc as plsc

_BEV_FEAT_SHAPE = (1, 1, 200, 200, 64)
_NC, _NS, _LANES = 2, 16, 16
_NW = _NC * _NS
_IC = 80
_CC = 25
_NBUF = 4


@functools.lru_cache(maxsize=None)
def _make_bevpool(n_cells, mx, C, p1, f1):
    del p1, f1
    kc = C // _LANES
    G = _CC * mx
    n_str = G // _IC
    cells_per_w = n_cells // _NW
    n_chunks = cells_per_w // _CC
    idx_rows = cells_per_w * mx // _IC

    mesh = plsc.VectorSubcoreMesh(
        core_axis_name="c", subcore_axis_name="s",
        num_cores=_NC, num_subcores=_NS)

    del idx_rows
    n_idx = cells_per_w * mx
    scratch = ([pltpu.VMEM((n_idx,), jnp.int32)] * 2
               + [pltpu.VMEM((G,), jnp.float32)] * _NBUF
               + [pltpu.VMEM((G, C), jnp.bfloat16)] * _NBUF
               + [pltpu.VMEM((_CC * C,), jnp.float32)] * 2
               + [pltpu.SemaphoreType.DMA] * 2)

    @functools.partial(
        pl.kernel,
        out_type=jax.ShapeDtypeStruct((n_cells * C,), jnp.float32),
        mesh=mesh,
        compiler_params=pltpu.CompilerParams(
            use_tc_tiling_on_sc=False, needs_layout_passes=False),
        scratch_types=scratch,
    )
    def bev_kernel(depth_hbm, feat_hbm, rd_hbm, rf_hbm, out_hbm,
                   rdx_v, rfx_v, dv0, dv1, dv2, dv3, fr0, fr1, fr2, fr3,
                   ob0, ob1, sem_g, sem_o):
        dvs = (dv0, dv1, dv2, dv3)
        frs = (fr0, fr1, fr2, fr3)
        obs = (ob0, ob1)
        wid = lax.axis_index("s") * _NC + lax.axis_index("c")
        cell_base = wid * cells_per_w
        idx_base = wid * n_idx

        pltpu.sync_copy(rd_hbm.at[pl.ds(idx_base, n_idx)], rdx_v)
        pltpu.sync_copy(rf_hbm.at[pl.ds(idx_base, n_idx)], rfx_v)

        def fire_gathers(t, dv, fr):
            for i in range(n_str):
                r = (t * n_str + i) * _IC
                pltpu.async_copy(
                    depth_hbm.at[rdx_v.at[pl.ds(r, _IC)]],
                    dv.at[pl.ds(i * _IC, _IC)], sem_g)
                pltpu.async_copy(
                    feat_hbm.at[rfx_v.at[pl.ds(r, _IC)]],
                    fr.at[pl.ds(i * _IC, _IC)], sem_g)

        def drain_gathers(dv, fr):
            for i in range(n_str):
                pltpu.make_async_copy(
                    depth_hbm.at[rdx_v.at[pl.ds(i * _IC, _IC)]],
                    dv.at[pl.ds(i * _IC, _IC)], sem_g).wait()
                pltpu.make_async_copy(
                    feat_hbm.at[rfx_v.at[pl.ds(i * _IC, _IC)]],
                    fr.at[pl.ds(i * _IC, _IC)], sem_g).wait()

        def compute_chunk(t, dv, fr, ob):
            def cell_body(j, _):
                pb = j * mx
                dvec = dv[pl.ds(pb, mx)]
                dd = plsc.bitcast(
                    plsc.pack(dvec, dvec, format=plsc.PackFormat.INTERLEAVED),
                    jnp.int32)
                acc = [jnp.zeros((_LANES,), jnp.float32) for _ in range(kc)]
                for p in range(mx):
                    dsplat = plsc.bitcast(
                        jnp.broadcast_to(dd[p], (_LANES,)), jnp.bfloat16)
                    for k2 in range(kc // 2):
                        packed = fr[pb + p, pl.ds(k2 * 2 * _LANES, 2 * _LANES)]
                        lo, hi = plsc.unpack(
                            dsplat * packed, format=plsc.PackFormat.INTERLEAVED)
                        acc[2 * k2] = acc[2 * k2] + lo
                        acc[2 * k2 + 1] = acc[2 * k2 + 1] + hi
                for k in range(kc):
                    ob[pl.ds(j * C + k * _LANES, _LANES)] = acc[k]
                return 0

            lax.fori_loop(0, _CC, cell_body, 0)
            pltpu.async_copy(
                ob, out_hbm.at[pl.ds((cell_base + t * _CC) * C, _CC * C)],
                sem_o)

        def drain_out(ob):
            pltpu.make_async_copy(
                ob, out_hbm.at[pl.ds(cell_base * C, _CC * C)], sem_o).wait()

        fire_gathers(0, dvs[0], frs[0])
        fire_gathers(1, dvs[1], frs[1])

        def quad_body(q, _):
            for u in range(_NBUF):
                t = _NBUF * q + u
                fire_gathers(t + 2, dvs[(u + 2) % _NBUF], frs[(u + 2) % _NBUF])
                drain_gathers(dvs[u], frs[u])
                if u >= 2:
                    drain_out(obs[u % 2])
                else:
                    @pl.when(t >= 2)
                    def _():
                        drain_out(obs[u % 2])
                compute_chunk(t, dvs[u], frs[u], obs[u % 2])
            return 0

        lax.fori_loop(0, (n_chunks - 2) // _NBUF, quad_body, 0)
        for tt in (n_chunks - 2, n_chunks - 1):
            u = tt % _NBUF
            drain_gathers(dvs[u], frs[u])
            drain_out(obs[tt % 2])
            compute_chunk(tt, dvs[u], frs[u], obs[tt % 2])
        drain_out(obs[0])
        drain_out(obs[1])

    return bev_kernel


def kernel(depth, feat, ranks_depth, ranks_feat, maxn):
    del maxn
    C = feat.shape[-1]
    _, oD, oW, oH, _ = _BEV_FEAT_SHAPE
    n_cells = oD * oW * oH
    L = ranks_depth.shape[0]
    mx = L // n_cells
    npad = 128 - depth.size % 128
    depth_flat = jnp.concatenate(
        [depth.reshape(-1), jnp.zeros((npad,), jnp.float32)])
    feat_2d = jnp.concatenate(
        [feat.reshape(-1, C), jnp.zeros((1, C), jnp.float32)], axis=0)
    f1 = feat_2d.shape[0]
    feat_bf = (feat_2d.reshape(f1, C // 32, 2, 16)
               .swapaxes(2, 3).reshape(f1, C).astype(jnp.bfloat16))
    fn = _make_bevpool(n_cells, mx, C, depth_flat.shape[0], f1)
    out = fn(depth_flat, feat_bf, ranks_depth, ranks_feat)
    return out.reshape(1, oD, oW, oH, C)


# speedup vs baseline: 18.0138x; 1.0018x over previous
import functools

import jax
import jax.numpy as jnp
from jax import lax
from jax.experimental import pallas as pl
from jax.experimental.pallas import tpu as pltpu
from jax.experimental.pallas import tpu_sc as plsc

_BEV_FEAT_SHAPE = (1, 1, 200, 200, 64)
_NC, _NS, _LANES = 2, 16, 16
_NW = _NC * _NS
_IC = 80
_CC = 50
_NBUF = 3


@functools.lru_cache(maxsize=None)
def _make_bevpool(n_cells, mx, C, p1, f1):
    del p1, f1
    kc = C // _LANES
    G = _CC * mx
    n_str = G // _IC
    cells_per_w = n_cells // _NW
    n_chunks = cells_per_w // _CC
    idx_rows = cells_per_w * mx // _IC

    mesh = plsc.VectorSubcoreMesh(
        core_axis_name="c", subcore_axis_name="s",
        num_cores=_NC, num_subcores=_NS)

    del idx_rows
    n_idx = cells_per_w * mx
    scratch = ([pltpu.VMEM((n_idx,), jnp.int32)] * 2
               + [pltpu.VMEM((G,), jnp.float32)] * _NBUF
               + [pltpu.VMEM((G, C), jnp.bfloat16)] * _NBUF
               + [pltpu.VMEM((_CC * C,), jnp.float32)] * _NBUF
               + [pltpu.SemaphoreType.DMA] * 2)

    @functools.partial(
        pl.kernel,
        out_type=jax.ShapeDtypeStruct((n_cells * C,), jnp.float32),
        mesh=mesh,
        compiler_params=pltpu.CompilerParams(
            use_tc_tiling_on_sc=False, needs_layout_passes=False),
        scratch_types=scratch,
    )
    def bev_kernel(depth_hbm, feat_hbm, rd_hbm, rf_hbm, out_hbm,
                   rdx_v, rfx_v, dv0, dv1, dv2, fr0, fr1, fr2,
                   ob0, ob1, ob2, sem_g, sem_o):
        dvs = (dv0, dv1, dv2)
        frs = (fr0, fr1, fr2)
        obs = (ob0, ob1, ob2)
        wid = lax.axis_index("s") * _NC + lax.axis_index("c")
        cell_base = wid * cells_per_w
        idx_base = wid * n_idx

        pltpu.sync_copy(rd_hbm.at[pl.ds(idx_base, n_idx)], rdx_v)
        pltpu.sync_copy(rf_hbm.at[pl.ds(idx_base, n_idx)], rfx_v)

        def fire_gathers(t, dv, fr):
            for i in range(n_str):
                r = (t * n_str + i) * _IC
                pltpu.async_copy(
                    depth_hbm.at[rdx_v.at[pl.ds(r, _IC)]],
                    dv.at[pl.ds(i * _IC, _IC)], sem_g)
                pltpu.async_copy(
                    feat_hbm.at[rfx_v.at[pl.ds(r, _IC)]],
                    fr.at[pl.ds(i * _IC, _IC)], sem_g)

        def drain_gathers(dv, fr):
            for i in range(n_str):
                pltpu.make_async_copy(
                    depth_hbm.at[rdx_v.at[pl.ds(i * _IC, _IC)]],
                    dv.at[pl.ds(i * _IC, _IC)], sem_g).wait()
                pltpu.make_async_copy(
                    feat_hbm.at[rfx_v.at[pl.ds(i * _IC, _IC)]],
                    fr.at[pl.ds(i * _IC, _IC)], sem_g).wait()

        def compute_chunk(t, dv, fr, ob):
            def cell_body(j, _):
                pb = j * mx
                dvec = dv[pl.ds(pb, mx)]
                dd = plsc.bitcast(
                    plsc.pack(dvec, dvec, format=plsc.PackFormat.INTERLEAVED),
                    jnp.int32)
                acc = [jnp.zeros((_LANES,), jnp.float32) for _ in range(kc)]
                for p in range(mx):
                    dsplat = plsc.bitcast(
                        jnp.broadcast_to(dd[p], (_LANES,)), jnp.bfloat16)
                    for k2 in range(kc // 2):
                        packed = fr[pb + p, pl.ds(k2 * 2 * _LANES, 2 * _LANES)]
                        lo, hi = plsc.unpack(
                            dsplat * packed, format=plsc.PackFormat.INTERLEAVED)
                        acc[2 * k2] = acc[2 * k2] + lo
                        acc[2 * k2 + 1] = acc[2 * k2 + 1] + hi
                for k in range(kc):
                    ob[pl.ds(j * C + k * _LANES, _LANES)] = acc[k]
                return 0

            lax.fori_loop(0, _CC, cell_body, 0)
            pltpu.async_copy(
                ob, out_hbm.at[pl.ds((cell_base + t * _CC) * C, _CC * C)],
                sem_o)

        def drain_out(ob):
            pltpu.make_async_copy(
                ob, out_hbm.at[pl.ds(cell_base * C, _CC * C)], sem_o).wait()

        fire_gathers(0, dvs[0], frs[0])
        fire_gathers(1, dvs[1], frs[1])

        n_main = n_chunks - 4

        def ring_body(q, _):
            for u in range(_NBUF):
                t = _NBUF * q + u
                fire_gathers(t + 2, dvs[(u + 2) % _NBUF], frs[(u + 2) % _NBUF])
                drain_gathers(dvs[u], frs[u])

                @pl.when(t >= _NBUF)
                def _():
                    drain_out(obs[u])

                compute_chunk(t, dvs[u], frs[u], obs[u])
            return 0

        lax.fori_loop(0, n_main // _NBUF, ring_body, 0)
        for tt in range(n_main, n_chunks):
            u = tt % _NBUF
            if tt + 2 < n_chunks:
                fire_gathers(tt + 2, dvs[(u + 2) % _NBUF], frs[(u + 2) % _NBUF])
            drain_gathers(dvs[u], frs[u])
            drain_out(obs[u])
            compute_chunk(tt, dvs[u], frs[u], obs[u])
        for tt in range(n_chunks - _NBUF, n_chunks):
            drain_out(obs[tt % _NBUF])

    return bev_kernel


def kernel(depth, feat, ranks_depth, ranks_feat, maxn):
    del maxn
    C = feat.shape[-1]
    _, oD, oW, oH, _ = _BEV_FEAT_SHAPE
    n_cells = oD * oW * oH
    L = ranks_depth.shape[0]
    mx = L // n_cells
    npad = 128 - depth.size % 128
    depth_flat = jnp.concatenate(
        [depth.reshape(-1), jnp.zeros((npad,), jnp.float32)])
    feat_2d = jnp.concatenate(
        [feat.reshape(-1, C), jnp.zeros((1, C), jnp.float32)], axis=0)
    f1 = feat_2d.shape[0]
    feat_bf = (feat_2d.reshape(f1, C // 32, 2, 16)
               .swapaxes(2, 3).reshape(f1, C).astype(jnp.bfloat16))
    fn = _make_bevpool(n_cells, mx, C, depth_flat.shape[0], f1)
    out = fn(depth_flat, feat_bf, ranks_depth, ranks_feat)
    return out.reshape(1, oD, oW, oH, C)
